# Initial kernel scaffold; baseline (speedup 1.0000x reference)
import functools

import jax
import jax.numpy as jnp
from jax import lax
from jax.experimental import pallas as pl
from jax.experimental.pallas import tpu as pltpu
from jax.experimental.pallas import tpu_sc as plsc

HEADS = 8
ROWW = 80
ADW = 16
CB = 128
NSC = 2
NSUB = 16
NW = NSC * NSUB


def _leaky(x):
    return jnp.maximum(x, 0.2 * x)


def _prep_body(x_ref, W_ref, As_ref, Ad_ref, T_ref, AD_ref, g_ref,
               asm, adm, *, last_step, hpad):
    B = x_ref.shape[0]
    h = jnp.dot(x_ref[...], W_ref[...], preferred_element_type=jnp.float32)
    a_s = jnp.dot(h, As_ref[...], preferred_element_type=jnp.float32)
    a_d = jnp.dot(h, Ad_ref[...], preferred_element_type=jnp.float32)
    z8 = jnp.zeros((B, 8), jnp.float32)
    if hpad:
        hp = jnp.concatenate([h, jnp.zeros((B, hpad), jnp.float32)], axis=1)
    else:
        hp = h
    T_ref[...] = jnp.concatenate([a_s, z8, hp], axis=1)
    AD_ref[...] = jnp.concatenate([a_d, z8], axis=1)
    step = pl.program_id(0)
    bs = jnp.max(a_s, axis=0, keepdims=True)
    bd = jnp.max(a_d, axis=0, keepdims=True)

    @pl.when(step == 0)
    def _():
        asm[...] = bs
        adm[...] = bd

    @pl.when(step != 0)
    def _():
        asm[...] = jnp.maximum(asm[...], bs)
        adm[...] = jnp.maximum(adm[...], bd)

    @pl.when(step == last_step)
    def _():
        g8 = _leaky(asm[...] + adm[...])
        g_ref[...] = jnp.concatenate([g8, jnp.zeros((1, 8), jnp.float32)],
                                     axis=1)


def _prep_tc(x, W, As, Ad, br):
    N, DIN = x.shape
    DOUT = W.shape[1]
    grid = N // br
    return pl.pallas_call(
        functools.partial(_prep_body, last_step=grid - 1, hpad=64 - DOUT),
        grid=(grid,),
        in_specs=[
            pl.BlockSpec((br, DIN), lambda i: (i, 0)),
            pl.BlockSpec((DIN, DOUT), lambda i: (0, 0)),
            pl.BlockSpec((DOUT, HEADS), lambda i: (0, 0)),
            pl.BlockSpec((DOUT, HEADS), lambda i: (0, 0)),
        ],
        out_specs=[
            pl.BlockSpec((br, ROWW), lambda i: (i, 0)),
            pl.BlockSpec((br, ADW), lambda i: (i, 0)),
            pl.BlockSpec((1, 16), lambda i: (0, 0)),
        ],
        out_shape=[
            jax.ShapeDtypeStruct((N, ROWW), jnp.float32),
            jax.ShapeDtypeStruct((N, ADW), jnp.float32),
            jax.ShapeDtypeStruct((1, 16), jnp.float32),
        ],
        scratch_shapes=[
            pltpu.VMEM((1, HEADS), jnp.float32),
            pltpu.VMEM((1, HEADS), jnp.float32),
        ],
    )(x, W, As, Ad)


def _norm_body(acc_ref, R_ref, b_ref, o_ref, *, C):
    den = acc_ref[0, :, 0:HEADS] + acc_ref[1, :, 0:HEADS]
    num = (acc_ref[0, :, 16:16 + HEADS * C] +
           acc_ref[1, :, 16:16 + HEADS * C])
    den_rep = jnp.dot(den, R_ref[...], preferred_element_type=jnp.float32)
    o_ref[...] = num / (den_rep + 1e-16) + b_ref[...]


def _norm_tc(acc, R, b, C, br):
    N = acc.shape[1]
    D = HEADS * C
    return pl.pallas_call(
        functools.partial(_norm_body, C=C),
        grid=(N // br,),
        in_specs=[
            pl.BlockSpec((2, br, ROWW), lambda i: (0, i, 0)),
            pl.BlockSpec((HEADS, D), lambda i: (0, 0)),
            pl.BlockSpec((1, D), lambda i: (0, 0)),
        ],
        out_specs=pl.BlockSpec((br, D), lambda i: (i, 0)),
        out_shape=jax.ShapeDtypeStruct((N, D), jnp.float32),
    )(acc, R, b)


def _mid_body(acc_ref, R_ref, b_ref, W_ref, As_ref, Ad_ref,
              T_ref, AD_ref, g_ref, asm, adm, *, last_step):
    den = acc_ref[0, :, 0:HEADS] + acc_ref[1, :, 0:HEADS]
    num = acc_ref[0, :, 16:80] + acc_ref[1, :, 16:80]
    B = den.shape[0]
    den_rep = jnp.dot(den, R_ref[...], preferred_element_type=jnp.float32)
    t = num / (den_rep + 1e-16) + b_ref[...]
    t = jnp.where(t > 0, t, jnp.exp(jnp.minimum(t, 0.0)) - 1.0)
    h = jnp.dot(t, W_ref[...], preferred_element_type=jnp.float32)
    a_s = jnp.dot(h, As_ref[...], preferred_element_type=jnp.float32)
    a_d = jnp.dot(h, Ad_ref[...], preferred_element_type=jnp.float32)
    z8 = jnp.zeros((B, 8), jnp.float32)
    T_ref[...] = jnp.concatenate([a_s, z8, h, z8], axis=1)
    AD_ref[...] = jnp.concatenate([a_d, z8], axis=1)
    step = pl.program_id(0)
    bs = jnp.max(a_s, axis=0, keepdims=True)
    bd = jnp.max(a_d, axis=0, keepdims=True)

    @pl.when(step == 0)
    def _():
        asm[...] = bs
        adm[...] = bd

    @pl.when(step != 0)
    def _():
        asm[...] = jnp.maximum(asm[...], bs)
        adm[...] = jnp.maximum(adm[...], bd)

    @pl.when(step == last_step)
    def _():
        g8 = _leaky(asm[...] + adm[...])
        g_ref[...] = jnp.concatenate([g8, jnp.zeros((1, 8), jnp.float32)],
                                     axis=1)


def _mid_tc(acc, R, b, W, As, Ad, br):
    N = acc.shape[1]
    D2 = W.shape[1]
    grid = N // br
    return pl.pallas_call(
        functools.partial(_mid_body, last_step=grid - 1),
        grid=(grid,),
        in_specs=[
            pl.BlockSpec((2, br, ROWW), lambda i: (0, i, 0)),
            pl.BlockSpec((HEADS, 64), lambda i: (0, 0)),
            pl.BlockSpec((1, 64), lambda i: (0, 0)),
            pl.BlockSpec((64, D2), lambda i: (0, 0)),
            pl.BlockSpec((D2, HEADS), lambda i: (0, 0)),
            pl.BlockSpec((D2, HEADS), lambda i: (0, 0)),
        ],
        out_specs=[
            pl.BlockSpec((br, ROWW), lambda i: (i, 0)),
            pl.BlockSpec((br, ADW), lambda i: (i, 0)),
            pl.BlockSpec((1, 16), lambda i: (0, 0)),
        ],
        out_shape=[
            jax.ShapeDtypeStruct((N, ROWW), jnp.float32),
            jax.ShapeDtypeStruct((N, ADW), jnp.float32),
            jax.ShapeDtypeStruct((1, 16), jnp.float32),
        ],
        scratch_shapes=[
            pltpu.VMEM((1, HEADS), jnp.float32),
            pltpu.VMEM((1, HEADS), jnp.float32),
        ],
    )(acc, R, b, W, As, Ad)


def _vsel(x, idx):
    dn = lax.GatherDimensionNumbers(
        offset_dims=(), collapsed_slice_dims=(0,), start_index_map=(0,))
    return lax.gather(x, idx.reshape(16, 1), dn, (1,),
                      mode=lax.GatherScatterMode.PROMISE_IN_BOUNDS)


def _edge_sc(T, AD, g, src, dst, *, C):
    N = T.shape[0]
    E = src.shape[0]
    nchunk = E // CB
    rps = N // NSUB
    zrows = rps // 5

    mesh = plsc.VectorSubcoreMesh(core_axis_name="c", subcore_axis_name="s")

    @functools.partial(
        pl.kernel,
        out_type=jax.ShapeDtypeStruct((NSC, N, ROWW), jnp.float32),
        mesh=mesh,
        scratch_types=[
            pltpu.VMEM((CB,), jnp.int32),
            pltpu.VMEM((CB,), jnp.int32),
            pltpu.VMEM((CB, ROWW), jnp.float32),
            pltpu.VMEM((CB, ADW), jnp.float32),
            pltpu.VMEM((CB, ROWW), jnp.float32),
            pltpu.VMEM((16,), jnp.float32),
            pltpu.VMEM((zrows, ROWW), jnp.float32),
            pltpu.VMEM_SHARED((N, ROWW), jnp.float32),
            pltpu.SemaphoreType.DMA,
            pltpu.SemaphoreType.DMA,
        ],
    )
    def ek(T_hbm, AD_hbm, g_hbm, src_hbm, dst_hbm, out_hbm,
           sidx, didx, srcrows, dstrows, outrows, gv, zbuf, acc, sem1, sem2):
        c = lax.axis_index("c")
        s = lax.axis_index("s")
        wid = s * NSC + c

        zv = jnp.zeros((16,), jnp.float32)

        @pl.loop(0, zrows)
        def _(r):
            for q in range(ROWW // 16):
                zbuf[r, pl.ds(q * 16, 16)] = zv

        for t in range(rps // zrows):
            pltpu.sync_copy(zbuf, acc.at[pl.ds(s * rps + t * zrows, zrows)])

        pltpu.sync_copy(g_hbm, gv)
        plsc.subcore_barrier()

        g16 = gv[...]
        iota = lax.iota(jnp.int32, 16)
        sel = [(iota + 16 * q) // C for q in range(4)]

        def chunk(k):
            eoff = k * CB
            pltpu.sync_copy(src_hbm.at[pl.ds(eoff, CB)], sidx)
            pltpu.sync_copy(dst_hbm.at[pl.ds(eoff, CB)], didx)
            cp1 = pltpu.async_copy(T_hbm.at[sidx], srcrows, sem1)
            cp2 = pltpu.async_copy(AD_hbm.at[didx], dstrows, sem2)
            cp1.wait()
            cp2.wait()

            @pl.loop(0, CB)
            def _(e):
                a = srcrows[e, pl.ds(0, 16)] + dstrows[e, pl.ds(0, 16)]
                p = jnp.exp(_leaky(a) - g16)
                outrows[e, pl.ds(0, 16)] = p
                for q in range(4):
                    hv = srcrows[e, pl.ds(16 + q * 16, 16)]
                    outrows[e, pl.ds(16 + q * 16, 16)] = hv * _vsel(p, sel[q])

            pltpu.sync_copy(outrows, acc.at[didx], add=True)

        count = (nchunk - wid + NW - 1) // NW

        def body(i, carry):
            chunk(wid + i * NW)
            return carry

        lax.fori_loop(0, count, body, 0)

        plsc.subcore_barrier()
        pltpu.sync_copy(acc.at[pl.ds(s * rps, rps)],
                        out_hbm.at[c, pl.ds(s * rps, rps)])

    return ek(T, AD, g, src, dst)


def _blockdiag(a):
    H, C = a.shape
    f = jnp.arange(H * C)
    return jnp.zeros((H * C, H), jnp.float32).at[f, f // C].set(a.reshape(-1))


def _repmat(C):
    return (jnp.arange(HEADS * C) // C ==
            jnp.arange(HEADS)[:, None]).astype(jnp.float32)


def kernel(x, edge_index, W1, a1_src, a1_dst, b1, W2, a2_src, a2_dst, b2):
    src = edge_index[0].astype(jnp.int32)
    dst = edge_index[1].astype(jnp.int32)
    br = 1000

    T1, AD1, g1 = _prep_tc(x, W1, _blockdiag(a1_src), _blockdiag(a1_dst), br)
    acc1 = _edge_sc(T1, AD1, g1.reshape(16), src, dst, C=8)
    T2, AD2, g2 = _mid_tc(acc1, _repmat(8), b1.reshape(1, -1), W2,
                          _blockdiag(a2_src), _blockdiag(a2_dst), br)
    acc2 = _edge_sc(T2, AD2, g2.reshape(16), src, dst, C=7)
    out = _norm_tc(acc2, _repmat(7), b2.reshape(1, -1), 7, br)
    return out


# speedup vs baseline: 115.6973x; 115.6973x over previous
import functools

import jax
import jax.numpy as jnp
from jax import lax
from jax.experimental import pallas as pl
from jax.experimental.pallas import tpu as pltpu
from jax.experimental.pallas import tpu_sc as plsc

HEADS = 8
ROWW = 80
ADW = 16
CB = 128
NSC = 2
NSUB = 16
NW = NSC * NSUB


def _leaky(x):
    return jnp.maximum(x, 0.2 * x)


def _prep_body(x_ref, W_ref, As_ref, Ad_ref, T_ref, AD_ref, g_ref,
               asm, adm, *, last_step, hpad):
    B = x_ref.shape[0]
    h = jnp.dot(x_ref[...], W_ref[...], preferred_element_type=jnp.float32)
    a_s = jnp.dot(h, As_ref[...], preferred_element_type=jnp.float32)
    a_d = jnp.dot(h, Ad_ref[...], preferred_element_type=jnp.float32)
    z8 = jnp.zeros((B, 8), jnp.float32)
    if hpad:
        hp = jnp.concatenate([h, jnp.zeros((B, hpad), jnp.float32)], axis=1)
    else:
        hp = h
    T_ref[...] = jnp.concatenate([a_s, z8, hp], axis=1)
    AD_ref[...] = jnp.concatenate([a_d, z8], axis=1)
    step = pl.program_id(0)
    bs = jnp.max(a_s, axis=0, keepdims=True)
    bd = jnp.max(a_d, axis=0, keepdims=True)

    @pl.when(step == 0)
    def _():
        asm[...] = bs
        adm[...] = bd

    @pl.when(step != 0)
    def _():
        asm[...] = jnp.maximum(asm[...], bs)
        adm[...] = jnp.maximum(adm[...], bd)

    @pl.when(step == last_step)
    def _():
        g8 = _leaky(asm[...] + adm[...])
        g_ref[...] = jnp.concatenate([g8, jnp.zeros((1, 8), jnp.float32)],
                                     axis=1)


def _prep_tc(x, W, As, Ad, br):
    N, DIN = x.shape
    DOUT = W.shape[1]
    grid = N // br
    return pl.pallas_call(
        functools.partial(_prep_body, last_step=grid - 1, hpad=64 - DOUT),
        grid=(grid,),
        in_specs=[
            pl.BlockSpec((br, DIN), lambda i: (i, 0)),
            pl.BlockSpec((DIN, DOUT), lambda i: (0, 0)),
            pl.BlockSpec((DOUT, HEADS), lambda i: (0, 0)),
            pl.BlockSpec((DOUT, HEADS), lambda i: (0, 0)),
        ],
        out_specs=[
            pl.BlockSpec((br, ROWW), lambda i: (i, 0)),
            pl.BlockSpec((br, ADW), lambda i: (i, 0)),
            pl.BlockSpec((1, 16), lambda i: (0, 0)),
        ],
        out_shape=[
            jax.ShapeDtypeStruct((N, ROWW), jnp.float32),
            jax.ShapeDtypeStruct((N, ADW), jnp.float32),
            jax.ShapeDtypeStruct((1, 16), jnp.float32),
        ],
        scratch_shapes=[
            pltpu.VMEM((1, HEADS), jnp.float32),
            pltpu.VMEM((1, HEADS), jnp.float32),
        ],
    )(x, W, As, Ad)


def _norm_body(acc_ref, R_ref, b_ref, o_ref, *, C):
    den = acc_ref[0, :, 0:HEADS] + acc_ref[1, :, 0:HEADS]
    num = (acc_ref[0, :, 16:16 + HEADS * C] +
           acc_ref[1, :, 16:16 + HEADS * C])
    den_rep = jnp.dot(den, R_ref[...], preferred_element_type=jnp.float32)
    o_ref[...] = num / (den_rep + 1e-16) + b_ref[...]


def _norm_tc(acc, R, b, C, br):
    N = acc.shape[1]
    D = HEADS * C
    return pl.pallas_call(
        functools.partial(_norm_body, C=C),
        grid=(N // br,),
        in_specs=[
            pl.BlockSpec((2, br, ROWW), lambda i: (0, i, 0)),
            pl.BlockSpec((HEADS, D), lambda i: (0, 0)),
            pl.BlockSpec((1, D), lambda i: (0, 0)),
        ],
        out_specs=pl.BlockSpec((br, D), lambda i: (i, 0)),
        out_shape=jax.ShapeDtypeStruct((N, D), jnp.float32),
    )(acc, R, b)


def _mid_body(acc_ref, R_ref, b_ref, W_ref, As_ref, Ad_ref,
              T_ref, AD_ref, g_ref, asm, adm, *, last_step):
    den = acc_ref[0, :, 0:HEADS] + acc_ref[1, :, 0:HEADS]
    num = acc_ref[0, :, 16:80] + acc_ref[1, :, 16:80]
    B = den.shape[0]
    den_rep = jnp.dot(den, R_ref[...], preferred_element_type=jnp.float32)
    t = num / (den_rep + 1e-16) + b_ref[...]
    t = jnp.where(t > 0, t, jnp.exp(jnp.minimum(t, 0.0)) - 1.0)
    h = jnp.dot(t, W_ref[...], preferred_element_type=jnp.float32)
    a_s = jnp.dot(h, As_ref[...], preferred_element_type=jnp.float32)
    a_d = jnp.dot(h, Ad_ref[...], preferred_element_type=jnp.float32)
    z8 = jnp.zeros((B, 8), jnp.float32)
    T_ref[...] = jnp.concatenate([a_s, z8, h, z8], axis=1)
    AD_ref[...] = jnp.concatenate([a_d, z8], axis=1)
    step = pl.program_id(0)
    bs = jnp.max(a_s, axis=0, keepdims=True)
    bd = jnp.max(a_d, axis=0, keepdims=True)

    @pl.when(step == 0)
    def _():
        asm[...] = bs
        adm[...] = bd

    @pl.when(step != 0)
    def _():
        asm[...] = jnp.maximum(asm[...], bs)
        adm[...] = jnp.maximum(adm[...], bd)

    @pl.when(step == last_step)
    def _():
        g8 = _leaky(asm[...] + adm[...])
        g_ref[...] = jnp.concatenate([g8, jnp.zeros((1, 8), jnp.float32)],
                                     axis=1)


def _mid_tc(acc, R, b, W, As, Ad, br):
    N = acc.shape[1]
    D2 = W.shape[1]
    grid = N // br
    return pl.pallas_call(
        functools.partial(_mid_body, last_step=grid - 1),
        grid=(grid,),
        in_specs=[
            pl.BlockSpec((2, br, ROWW), lambda i: (0, i, 0)),
            pl.BlockSpec((HEADS, 64), lambda i: (0, 0)),
            pl.BlockSpec((1, 64), lambda i: (0, 0)),
            pl.BlockSpec((64, D2), lambda i: (0, 0)),
            pl.BlockSpec((D2, HEADS), lambda i: (0, 0)),
            pl.BlockSpec((D2, HEADS), lambda i: (0, 0)),
        ],
        out_specs=[
            pl.BlockSpec((br, ROWW), lambda i: (i, 0)),
            pl.BlockSpec((br, ADW), lambda i: (i, 0)),
            pl.BlockSpec((1, 16), lambda i: (0, 0)),
        ],
        out_shape=[
            jax.ShapeDtypeStruct((N, ROWW), jnp.float32),
            jax.ShapeDtypeStruct((N, ADW), jnp.float32),
            jax.ShapeDtypeStruct((1, 16), jnp.float32),
        ],
        scratch_shapes=[
            pltpu.VMEM((1, HEADS), jnp.float32),
            pltpu.VMEM((1, HEADS), jnp.float32),
        ],
    )(acc, R, b, W, As, Ad)


def _vsel(x, idx):
    dn = lax.GatherDimensionNumbers(
        offset_dims=(), collapsed_slice_dims=(0,), start_index_map=(0,))
    return lax.gather(x, idx.reshape(16, 1), dn, (1,),
                      mode=lax.GatherScatterMode.PROMISE_IN_BOUNDS)


def _edge_sc(T, AD, g, src, dst, *, C):
    N = T.shape[0]
    E = src.shape[0]
    nchunk = E // CB
    rps = (N // NSUB) // 8 * 8
    extra = N - rps * NSUB
    zrows = rps // 3

    mesh = plsc.VectorSubcoreMesh(core_axis_name="c", subcore_axis_name="s")

    @functools.partial(
        pl.kernel,
        out_type=jax.ShapeDtypeStruct((NSC, N, ROWW), jnp.float32),
        mesh=mesh,
        compiler_params=pltpu.CompilerParams(needs_layout_passes=False,
                                             use_tc_tiling_on_sc=False),
        scratch_types=[
            pltpu.VMEM((CB,), jnp.int32),
            pltpu.VMEM((CB,), jnp.int32),
            pltpu.VMEM((CB, ROWW), jnp.float32),
            pltpu.VMEM((CB, ADW), jnp.float32),
            pltpu.VMEM((CB, ROWW), jnp.float32),
            pltpu.VMEM((16,), jnp.float32),
            pltpu.VMEM((zrows, ROWW), jnp.float32),
            pltpu.VMEM_SHARED((N, ROWW), jnp.float32),
            pltpu.SemaphoreType.DMA,
            pltpu.SemaphoreType.DMA,
        ],
    )
    def ek(T_hbm, AD_hbm, g_hbm, src_hbm, dst_hbm, out_hbm,
           sidx, didx, srcrows, dstrows, outrows, gv, zbuf, acc, sem1, sem2):
        c = lax.axis_index("c")
        s = lax.axis_index("s")
        wid = s * NSC + c

        zv = jnp.zeros((16,), jnp.float32)

        @pl.loop(0, zrows)
        def _(r):
            for q in range(ROWW // 16):
                zbuf[r, pl.ds(q * 16, 16)] = zv

        for t in range(rps // zrows):
            pltpu.sync_copy(zbuf, acc.at[pl.ds(s * rps + t * zrows, zrows)])

        @pl.when(s == NSUB - 1)
        def _():
            pltpu.sync_copy(zbuf.at[pl.ds(0, extra)],
                            acc.at[pl.ds(NSUB * rps, extra)])

        pltpu.sync_copy(g_hbm, gv)
        plsc.subcore_barrier()

        g16 = gv[...]
        iota = lax.iota(jnp.int32, 16)
        sel = [(iota + 16 * q) // C for q in range(4)]

        def chunk(k):
            eoff = k * CB
            pltpu.sync_copy(src_hbm.at[pl.ds(eoff, CB)], sidx)
            pltpu.sync_copy(dst_hbm.at[pl.ds(eoff, CB)], didx)
            cp1 = pltpu.async_copy(T_hbm.at[sidx], srcrows, sem1)
            cp2 = pltpu.async_copy(AD_hbm.at[didx], dstrows, sem2)
            cp1.wait()
            cp2.wait()

            @pl.loop(0, CB)
            def _(e):
                a = srcrows[e, pl.ds(0, 16)] + dstrows[e, pl.ds(0, 16)]
                p = jnp.exp(_leaky(a) - g16)
                outrows[e, pl.ds(0, 16)] = p
                for q in range(4):
                    hv = srcrows[e, pl.ds(16 + q * 16, 16)]
                    outrows[e, pl.ds(16 + q * 16, 16)] = hv * _vsel(p, sel[q])

            pltpu.sync_copy(outrows, acc.at[didx], add=True)

        count = (nchunk - wid + NW - 1) // NW

        def body(i, carry):
            chunk(wid + i * NW)
            return carry

        lax.fori_loop(0, count, body, 0)

        plsc.subcore_barrier()
        pltpu.sync_copy(acc.at[pl.ds(s * rps, rps)],
                        out_hbm.at[c, pl.ds(s * rps, rps)])

        @pl.when(s == NSUB - 1)
        def _():
            pltpu.sync_copy(acc.at[pl.ds(NSUB * rps, extra)],
                            out_hbm.at[c, pl.ds(NSUB * rps, extra)])

    return ek(T, AD, g, src, dst)


def _blockdiag(a):
    H, C = a.shape
    f = jnp.arange(H * C)
    return jnp.zeros((H * C, H), jnp.float32).at[f, f // C].set(a.reshape(-1))


def _repmat(C):
    return (jnp.arange(HEADS * C) // C ==
            jnp.arange(HEADS)[:, None]).astype(jnp.float32)


def kernel(x, edge_index, W1, a1_src, a1_dst, b1, W2, a2_src, a2_dst, b2):
    src = edge_index[0].astype(jnp.int32)
    dst = edge_index[1].astype(jnp.int32)
    br = 1000

    T1, AD1, g1 = _prep_tc(x, W1, _blockdiag(a1_src), _blockdiag(a1_dst), br)
    acc1 = _edge_sc(T1, AD1, g1.reshape(16), src, dst, C=8)
    T2, AD2, g2 = _mid_tc(acc1, _repmat(8), b1.reshape(1, -1), W2,
                          _blockdiag(a2_src), _blockdiag(a2_dst), br)
    acc2 = _edge_sc(T2, AD2, g2.reshape(16), src, dst, C=7)
    out = _norm_tc(acc2, _repmat(7), b2.reshape(1, -1), 7, br)
    return out
